# Initial kernel scaffold; baseline (speedup 1.0000x reference)
import jax
import jax.numpy as jnp
from jax.experimental import pallas as pl


def kernel(x, W):
    raise NotImplementedError("write your pallas kernel here")


# speedup vs baseline: 1.3500x; 1.3500x over previous
import functools

import jax
import jax.numpy as jnp
from jax import lax
from jax.experimental import pallas as pl
from jax.experimental.pallas import tpu as pltpu
from jax.experimental.pallas import tpu_sc as plsc

_DIM = 2048
_NE = 8
_K = 2
_T = 32768
_BT = 512

_NC = 2
_NS = 16
_NW = _NC * _NS
_CHUNK = _T // _NW
_L = 16


def _tc_scores_body(w_ref, x_ref, o_ref):
    o_ref[...] = lax.dot_general(
        w_ref[...], x_ref[...],
        (((1,), (1,)), ((), ())),
        preferred_element_type=jnp.float32,
    )


def _tc_scores(x, W):
    return pl.pallas_call(
        _tc_scores_body,
        grid=(_T // _BT,),
        in_specs=[
            pl.BlockSpec((_NE, _DIM), lambda i: (0, 0)),
            pl.BlockSpec((_BT, _DIM), lambda i: (i, 0)),
        ],
        out_specs=pl.BlockSpec((_NE, _BT), lambda i: (0, i)),
        out_shape=jax.ShapeDtypeStruct((_NE, _T), jnp.float32),
    )(W, x)


@functools.partial(
    pl.kernel,
    mesh=plsc.VectorSubcoreMesh(core_axis_name="c", subcore_axis_name="s"),
    out_type=[
        jax.ShapeDtypeStruct((_T,), jnp.int32),
        jax.ShapeDtypeStruct((_T,), jnp.int32),
        jax.ShapeDtypeStruct((_T,), jnp.float32),
        jax.ShapeDtypeStruct((_T,), jnp.float32),
    ],
    scratch_types=[
        pltpu.VMEM((_NE, _CHUNK), jnp.float32),
        pltpu.VMEM((_CHUNK,), jnp.int32),
        pltpu.VMEM((_CHUNK,), jnp.int32),
        pltpu.VMEM((_CHUNK,), jnp.float32),
        pltpu.VMEM((_CHUNK,), jnp.float32),
    ],
)
def _sc_route(scores_hbm, i1_hbm, i2_hbm, w1_hbm, w2_hbm,
              s_v, i1_v, i2_v, w1_v, w2_v):
    wid = lax.axis_index("s") * _NC + lax.axis_index("c")
    base = wid * _CHUNK
    pltpu.sync_copy(scores_hbm.at[:, pl.ds(base, _CHUNK)], s_v)

    neg_inf = jnp.full((_L,), -jnp.inf, jnp.float32)

    def body(i, carry):
        off = i * _L
        s = [s_v[k, pl.ds(off, _L)] for k in range(_NE)]
        m1 = s[0]
        for k in range(1, _NE):
            m1 = jnp.maximum(m1, s[k])
        i1 = jnp.full((_L,), _NE - 1, jnp.int32)
        for k in range(_NE - 2, -1, -1):
            i1 = jnp.where(s[k] == m1, jnp.full((_L,), k, jnp.int32), i1)
        s2 = [jnp.where(i1 == k, neg_inf, s[k]) for k in range(_NE)]
        m2 = s2[0]
        for k in range(1, _NE):
            m2 = jnp.maximum(m2, s2[k])
        i2 = jnp.full((_L,), _NE - 1, jnp.int32)
        for k in range(_NE - 2, -1, -1):
            i2 = jnp.where(s2[k] == m2, jnp.full((_L,), k, jnp.int32), i2)
        i1_v[pl.ds(off, _L)] = i1
        i2_v[pl.ds(off, _L)] = i2
        w1_v[pl.ds(off, _L)] = m1
        w2_v[pl.ds(off, _L)] = m2
        return carry

    lax.fori_loop(0, _CHUNK // _L, body, 0)
    pltpu.sync_copy(i1_v, i1_hbm.at[pl.ds(base, _CHUNK)])
    pltpu.sync_copy(i2_v, i2_hbm.at[pl.ds(base, _CHUNK)])
    pltpu.sync_copy(w1_v, w1_hbm.at[pl.ds(base, _CHUNK)])
    pltpu.sync_copy(w2_v, w2_hbm.at[pl.ds(base, _CHUNK)])


def kernel(x, W):
    scores_t = _tc_scores(x, W)
    i1, i2, w1, w2 = _sc_route(scores_t)
    return jnp.stack([i1, i2], axis=1), jnp.stack([w1, w2], axis=1)


# speedup vs baseline: 1.5346x; 1.1367x over previous
import functools

import jax
import jax.numpy as jnp
from jax import lax
from jax.experimental import pallas as pl
from jax.experimental.pallas import tpu as pltpu
from jax.experimental.pallas import tpu_sc as plsc

_DIM = 2048
_NE = 8
_K = 2
_T = 32768
_BT = 2048

_NC = 2
_NS = 16
_NW = _NC * _NS
_CHUNK = _T // _NW
_L = 16


def _tc_scores_body(w_ref, x_ref, o_ref):
    o_ref[...] = lax.dot_general(
        w_ref[...], x_ref[...],
        (((1,), (1,)), ((), ())),
        preferred_element_type=jnp.float32,
    )


def _tc_scores(x, W):
    return pl.pallas_call(
        _tc_scores_body,
        grid=(_T // _BT,),
        in_specs=[
            pl.BlockSpec((_NE, _DIM), lambda i: (0, 0)),
            pl.BlockSpec((_BT, _DIM), lambda i: (i, 0)),
        ],
        out_specs=pl.BlockSpec((_NE, _BT), lambda i: (0, i)),
        out_shape=jax.ShapeDtypeStruct((_NE, _T), jnp.float32),
    )(W, x)


@functools.partial(
    pl.kernel,
    mesh=plsc.VectorSubcoreMesh(core_axis_name="c", subcore_axis_name="s"),
    out_type=[
        jax.ShapeDtypeStruct((_T,), jnp.int32),
        jax.ShapeDtypeStruct((_T,), jnp.int32),
        jax.ShapeDtypeStruct((_T,), jnp.float32),
        jax.ShapeDtypeStruct((_T,), jnp.float32),
    ],
    scratch_types=[
        pltpu.VMEM((_NE, _CHUNK), jnp.float32),
        pltpu.VMEM((_CHUNK,), jnp.int32),
        pltpu.VMEM((_CHUNK,), jnp.int32),
        pltpu.VMEM((_CHUNK,), jnp.float32),
        pltpu.VMEM((_CHUNK,), jnp.float32),
    ],
)
def _sc_route(scores_hbm, i1_hbm, i2_hbm, w1_hbm, w2_hbm,
              s_v, i1_v, i2_v, w1_v, w2_v):
    wid = lax.axis_index("s") * _NC + lax.axis_index("c")
    base = wid * _CHUNK
    pltpu.sync_copy(scores_hbm.at[:, pl.ds(base, _CHUNK)], s_v)

    neg_inf = jnp.full((_L,), -jnp.inf, jnp.float32)

    def body(i, carry):
        off = i * _L
        s = [s_v[k, pl.ds(off, _L)] for k in range(_NE)]
        m1 = s[0]
        for k in range(1, _NE):
            m1 = jnp.maximum(m1, s[k])
        i1 = jnp.full((_L,), _NE - 1, jnp.int32)
        for k in range(_NE - 2, -1, -1):
            i1 = jnp.where(s[k] == m1, jnp.full((_L,), k, jnp.int32), i1)
        s2 = [jnp.where(i1 == k, neg_inf, s[k]) for k in range(_NE)]
        m2 = s2[0]
        for k in range(1, _NE):
            m2 = jnp.maximum(m2, s2[k])
        i2 = jnp.full((_L,), _NE - 1, jnp.int32)
        for k in range(_NE - 2, -1, -1):
            i2 = jnp.where(s2[k] == m2, jnp.full((_L,), k, jnp.int32), i2)
        i1_v[pl.ds(off, _L)] = i1
        i2_v[pl.ds(off, _L)] = i2
        w1_v[pl.ds(off, _L)] = m1
        w2_v[pl.ds(off, _L)] = m2
        return carry

    lax.fori_loop(0, _CHUNK // _L, body, 0)
    pltpu.sync_copy(i1_v, i1_hbm.at[pl.ds(base, _CHUNK)])
    pltpu.sync_copy(i2_v, i2_hbm.at[pl.ds(base, _CHUNK)])
    pltpu.sync_copy(w1_v, w1_hbm.at[pl.ds(base, _CHUNK)])
    pltpu.sync_copy(w2_v, w2_hbm.at[pl.ds(base, _CHUNK)])


def kernel(x, W):
    scores_t = _tc_scores(x, W)
    i1, i2, w1, w2 = _sc_route(scores_t)
    return jnp.stack([i1, i2], axis=1), jnp.stack([w1, w2], axis=1)


# speedup vs baseline: 1.5614x; 1.0175x over previous
import functools

import jax
import jax.numpy as jnp
from jax import lax
from jax.experimental import pallas as pl
from jax.experimental.pallas import tpu as pltpu
from jax.experimental.pallas import tpu_sc as plsc

_DIM = 2048
_NE = 8
_K = 2
_T = 32768
_BT = 1024

_NC = 2
_NS = 16
_NW = _NC * _NS
_CHUNK = _T // _NW
_L = 16


def _tc_scores_body(w_ref, x_ref, o_ref):
    o_ref[...] = lax.dot_general(
        w_ref[...], x_ref[...],
        (((1,), (1,)), ((), ())),
        preferred_element_type=jnp.float32,
    )


def _tc_scores(x, W):
    return pl.pallas_call(
        _tc_scores_body,
        grid=(_T // _BT,),
        in_specs=[
            pl.BlockSpec((_NE, _DIM), lambda i: (0, 0)),
            pl.BlockSpec((_BT, _DIM), lambda i: (i, 0)),
        ],
        out_specs=pl.BlockSpec((_NE, _BT), lambda i: (0, i)),
        out_shape=jax.ShapeDtypeStruct((_NE, _T), jnp.float32),
    )(W, x)


@functools.partial(
    pl.kernel,
    mesh=plsc.VectorSubcoreMesh(core_axis_name="c", subcore_axis_name="s"),
    out_type=[
        jax.ShapeDtypeStruct((_T,), jnp.int32),
        jax.ShapeDtypeStruct((_T,), jnp.int32),
        jax.ShapeDtypeStruct((_T,), jnp.float32),
        jax.ShapeDtypeStruct((_T,), jnp.float32),
    ],
    scratch_types=[
        pltpu.VMEM((_NE, _CHUNK), jnp.float32),
        pltpu.VMEM((_CHUNK,), jnp.int32),
        pltpu.VMEM((_CHUNK,), jnp.int32),
        pltpu.VMEM((_CHUNK,), jnp.float32),
        pltpu.VMEM((_CHUNK,), jnp.float32),
    ],
)
def _sc_route(scores_hbm, i1_hbm, i2_hbm, w1_hbm, w2_hbm,
              s_v, i1_v, i2_v, w1_v, w2_v):
    wid = lax.axis_index("s") * _NC + lax.axis_index("c")
    base = wid * _CHUNK
    pltpu.sync_copy(scores_hbm.at[:, pl.ds(base, _CHUNK)], s_v)

    neg_inf = jnp.full((_L,), -jnp.inf, jnp.float32)

    def body(i, carry):
        off = i * _L
        s = [s_v[k, pl.ds(off, _L)] for k in range(_NE)]
        m1 = s[0]
        for k in range(1, _NE):
            m1 = jnp.maximum(m1, s[k])
        i1 = jnp.full((_L,), _NE - 1, jnp.int32)
        for k in range(_NE - 2, -1, -1):
            i1 = jnp.where(s[k] == m1, jnp.full((_L,), k, jnp.int32), i1)
        s2 = [jnp.where(i1 == k, neg_inf, s[k]) for k in range(_NE)]
        m2 = s2[0]
        for k in range(1, _NE):
            m2 = jnp.maximum(m2, s2[k])
        i2 = jnp.full((_L,), _NE - 1, jnp.int32)
        for k in range(_NE - 2, -1, -1):
            i2 = jnp.where(s2[k] == m2, jnp.full((_L,), k, jnp.int32), i2)
        i1_v[pl.ds(off, _L)] = i1
        i2_v[pl.ds(off, _L)] = i2
        w1_v[pl.ds(off, _L)] = m1
        w2_v[pl.ds(off, _L)] = m2
        return carry

    lax.fori_loop(0, _CHUNK // _L, body, 0)
    pltpu.sync_copy(i1_v, i1_hbm.at[pl.ds(base, _CHUNK)])
    pltpu.sync_copy(i2_v, i2_hbm.at[pl.ds(base, _CHUNK)])
    pltpu.sync_copy(w1_v, w1_hbm.at[pl.ds(base, _CHUNK)])
    pltpu.sync_copy(w2_v, w2_hbm.at[pl.ds(base, _CHUNK)])


def kernel(x, W):
    scores_t = _tc_scores(x, W)
    i1, i2, w1, w2 = _sc_route(scores_t)
    return jnp.stack([i1, i2], axis=1), jnp.stack([w1, w2], axis=1)
